# Initial kernel scaffold; baseline (speedup 1.0000x reference)
import jax
import jax.numpy as jnp
from jax.experimental import pallas as pl


def kernel(q_pts, s_pts, neighb_inds, x, q_lrf, s_lrf, weights, lrf_W, lrf_b, bias):
    raise NotImplementedError("write your pallas kernel here")


# speedup vs baseline: 1.2698x; 1.2698x over previous
import functools
import numpy as np
import jax
import jax.numpy as jnp
from jax import lax
from jax.experimental import pallas as pl
from jax.experimental.pallas import tpu as pltpu
from jax.experimental.pallas import tpu_sc as plsc

K = 15
P = 3
IN_CH = 128
NQ = 2
NS = 2
KP_EXTENT = 1.2
RADIUS = 2.5
N = 10000
H = 16
TBL_C = 256
NW = 32
CHUNK = 128
BPW = 5120
E_PAD = NW * BPW
NCHUNK = BPW // CHUNK
B = 200
QD_C = 21 + 3 * 36


def _kp_np():
    rng = np.random.RandomState(42)
    kp = rng.uniform(-1.0, 1.0, size=(K, P)).astype(np.float32)
    kp = kp / np.maximum(np.linalg.norm(kp, axis=1, keepdims=True), 1e-6)
    kp = kp * rng.uniform(0.3, 1.0, size=(K, 1)).astype(np.float32) * RADIUS
    kp[0, :] = 0.0
    return kp


_KP = _kp_np()
_KPR = [np.concatenate([_KP[:, r], _KP[:, r]]).astype(np.float32) for r in range(P)]


def _sc_gather(table, idx):
    mesh = plsc.VectorSubcoreMesh(core_axis_name="c", subcore_axis_name="s")

    @functools.partial(
        pl.kernel,
        mesh=mesh,
        out_type=jax.ShapeDtypeStruct((E_PAD, TBL_C), jnp.float32),
        scratch_types=[
            pltpu.VMEM((CHUNK,), jnp.int32),
            pltpu.VMEM((CHUNK, TBL_C), jnp.float32),
            pltpu.SemaphoreType.DMA,
        ],
    )
    def gk(table_hbm, idx_hbm, out_hbm, idx_v, rows_v, sem):
        wid = lax.axis_index("s") * 2 + lax.axis_index("c")
        base = wid * BPW

        def body(j, carry):
            off = base + j * CHUNK
            pltpu.sync_copy(idx_hbm.at[pl.ds(off, CHUNK)], idx_v)
            pltpu.async_copy(table_hbm.at[idx_v], rows_v, sem).wait()
            pltpu.sync_copy(rows_v, out_hbm.at[pl.ds(off, CHUNK)])
            return carry

        lax.fori_loop(0, NCHUNK, body, 0)

    return gk(table, idx)


def _tc_body(gh_ref, qd_ref, wcat_ref, bvec_ref, bias_ref, kpr_ref, o_ref):
    gh = gh_ref[...]
    qd = qd_ref[...]
    d = gh[:, :, 146:149] - qd[None, :, 0:3]
    als = []
    for q in range(NQ):
        a = d[:, :, 0:1] * qd[None, :, 3 + q * 9 + 0:3 + q * 9 + 3]
        for p in range(1, P):
            a = a + d[:, :, p:p + 1] * qd[None, :, 3 + q * 9 + p * 3:3 + q * 9 + (p + 1) * 3]
        als.append(a)
    al = jnp.concatenate(als, axis=2)
    sq = None
    for r in range(P):
        alx = jnp.concatenate(
            [jnp.broadcast_to(al[:, :, r:r + 1], (H, B, K)),
             jnp.broadcast_to(al[:, :, 3 + r:4 + r], (H, B, K))], axis=2)
        diff = alx - kpr_ref[...][r:r + 1, :][None]
        sq = diff * diff if sq is None else sq + diff * diff
    w = jnp.maximum(1.0 - jnp.sqrt(sq) * (1.0 / KP_EXTENT), 0.0)
    aw = w[:, :, :K] + w[:, :, K:]
    S = gh[:, :, 128:146]
    alrf = None
    for p in range(P):
        pieces = []
        for s in range(NS):
            sl = S[:, :, s * 9 + p * 3:s * 9 + (p + 1) * 3]
            pieces += [sl, sl, sl]
        e18 = jnp.concatenate(pieces, axis=2)
        sbig = jnp.concatenate([e18, e18], axis=2)
        rbig = qd[None, :, 21 + p * 36:21 + (p + 1) * 36]
        alrf = rbig * sbig if alrf is None else alrf + rbig * sbig
    feats = jnp.concatenate([gh[:, :, 0:128], alrf], axis=2)
    acc = jnp.dot(jnp.sum(aw, axis=0), bvec_ref[...],
                  preferred_element_type=jnp.float32)
    for k in range(K):
        wk = aw[:, :, k:k + 1] * feats
        wks = jnp.sum(wk, axis=0)
        acc = acc + jnp.dot(wks, wcat_ref[k], preferred_element_type=jnp.float32)
    outv = acc + bias_ref[...]
    o_ref[...] = jnp.where(outv >= 0.0, outv, 0.1 * outv)


def kernel(q_pts, s_pts, neighb_inds, x, q_lrf, s_lrf, weights, lrf_W, lrf_b, bias):
    table = jnp.concatenate(
        [x, s_lrf.reshape(N, NS * 9), s_pts,
         jnp.zeros((N, TBL_C - IN_CH - NS * 9 - P), jnp.float32)], axis=1)
    flat_idx = neighb_inds.T.reshape(-1).astype(jnp.int32)
    flat_idx = jnp.concatenate(
        [flat_idx, jnp.zeros((E_PAD - N * H,), jnp.int32)])

    rl = q_lrf.reshape(N, NQ * 9)
    rbigs = [jnp.broadcast_to(q_lrf[:, :, None, p, :, None],
                              (N, NQ, NS, P, P)).reshape(N, 36)
             for p in range(P)]
    qd2 = jnp.concatenate([q_pts, rl] + rbigs, axis=1)

    bw = jnp.zeros((2 * NS * 9, IN_CH), jnp.float32)
    bw = bw.at[0:18, 0:64].set(lrf_W).at[18:36, 64:128].set(lrf_W)
    wbot = weights[:, IN_CH:, :]
    wcat = jnp.concatenate(
        [weights[:, :IN_CH, :], jnp.einsum('fe,keo->kfo', bw, wbot)], axis=1)
    b2 = jnp.concatenate([lrf_b, lrf_b])
    bvec = jnp.einsum('e,keo->ko', b2, wbot)

    g = _sc_gather(table, flat_idx)
    gh = g[:N * H].reshape(H, N, TBL_C)

    out = pl.pallas_call(
        _tc_body,
        grid=(N // B,),
        in_specs=[
            pl.BlockSpec((H, B, TBL_C), lambda i: (0, i, 0)),
            pl.BlockSpec((B, QD_C), lambda i: (i, 0)),
            pl.BlockSpec((K, 164, 128), lambda i: (0, 0, 0)),
            pl.BlockSpec((K, 128), lambda i: (0, 0)),
            pl.BlockSpec((1, 128), lambda i: (0, 0)),
            pl.BlockSpec((P, 2 * K), lambda i: (0, 0)),
        ],
        out_specs=pl.BlockSpec((B, 128), lambda i: (i, 0)),
        out_shape=jax.ShapeDtypeStruct((N, 128), jnp.float32),
    )(gh, qd2, wcat, bvec, bias[None, :], jnp.asarray(np.stack(_KPR)))
    return out


# speedup vs baseline: 1.4814x; 1.1666x over previous
import functools
import numpy as np
import jax
import jax.numpy as jnp
from jax import lax
from jax.experimental import pallas as pl
from jax.experimental.pallas import tpu as pltpu
from jax.experimental.pallas import tpu_sc as plsc

K = 15
P = 3
IN_CH = 128
NQ = 2
NS = 2
KP_EXTENT = 1.2
RADIUS = 2.5
N = 10000
H = 16
TBL_C = 256
NW = 32
CHUNK = 128
N_PAD = 10240
BPW = 5120
E_PAD = NW * BPW
NCHUNK = BPW // CHUNK
B = 200
QD_C = 21 + 3 * 36


def _kp_np():
    rng = np.random.RandomState(42)
    kp = rng.uniform(-1.0, 1.0, size=(K, P)).astype(np.float32)
    kp = kp / np.maximum(np.linalg.norm(kp, axis=1, keepdims=True), 1e-6)
    kp = kp * rng.uniform(0.3, 1.0, size=(K, 1)).astype(np.float32) * RADIUS
    kp[0, :] = 0.0
    return kp


_KP = _kp_np()
_KPR = [np.concatenate([_KP[:, r], _KP[:, r]]).astype(np.float32) for r in range(P)]


def _sc_gather(table, idx):
    mesh = plsc.VectorSubcoreMesh(core_axis_name="c", subcore_axis_name="s")

    @functools.partial(
        pl.kernel,
        mesh=mesh,
        out_type=jax.ShapeDtypeStruct((E_PAD, TBL_C), jnp.float32),
        scratch_types=[
            pltpu.VMEM((CHUNK,), jnp.int32),
            pltpu.VMEM((CHUNK, TBL_C), jnp.float32),
            pltpu.SemaphoreType.DMA,
        ],
    )
    def gk(table_hbm, idx_hbm, out_hbm, idx_v, rows_v, sem):
        wid = lax.axis_index("s") * 2 + lax.axis_index("c")
        base = wid * BPW

        def body(j, carry):
            off = base + j * CHUNK
            pltpu.sync_copy(idx_hbm.at[pl.ds(off, CHUNK)], idx_v)
            pltpu.async_copy(table_hbm.at[idx_v], rows_v, sem).wait()
            pltpu.sync_copy(rows_v, out_hbm.at[pl.ds(off, CHUNK)])
            return carry

        lax.fori_loop(0, NCHUNK, body, 0)

    return gk(table, idx)


def _tc_body(gh_ref, qd_ref, wcat_ref, bvec_ref, bias_ref, kpr_ref, o_ref):
    gh = gh_ref[...]
    qd = qd_ref[...]
    d = gh[:, :, 146:149] - qd[None, :, 0:3]
    als = []
    for q in range(NQ):
        a = d[:, :, 0:1] * qd[None, :, 3 + q * 9 + 0:3 + q * 9 + 3]
        for p in range(1, P):
            a = a + d[:, :, p:p + 1] * qd[None, :, 3 + q * 9 + p * 3:3 + q * 9 + (p + 1) * 3]
        als.append(a)
    al = jnp.concatenate(als, axis=2)
    sq = None
    for r in range(P):
        alx = jnp.concatenate(
            [jnp.broadcast_to(al[:, :, r:r + 1], (H, B, K)),
             jnp.broadcast_to(al[:, :, 3 + r:4 + r], (H, B, K))], axis=2)
        diff = alx - kpr_ref[...][r:r + 1, :][None]
        sq = diff * diff if sq is None else sq + diff * diff
    w = jnp.maximum(1.0 - jnp.sqrt(sq) * (1.0 / KP_EXTENT), 0.0)
    aw = w[:, :, :K] + w[:, :, K:]
    S = gh[:, :, 128:146]
    alrf = None
    for p in range(P):
        pieces = []
        for s in range(NS):
            sl = S[:, :, s * 9 + p * 3:s * 9 + (p + 1) * 3]
            pieces += [sl, sl, sl]
        e18 = jnp.concatenate(pieces, axis=2)
        sbig = jnp.concatenate([e18, e18], axis=2)
        rbig = qd[None, :, 21 + p * 36:21 + (p + 1) * 36]
        alrf = rbig * sbig if alrf is None else alrf + rbig * sbig
    feats = jnp.concatenate([gh[:, :, 0:128], alrf], axis=2)
    acc = jnp.dot(jnp.sum(aw, axis=0), bvec_ref[...],
                  preferred_element_type=jnp.float32)
    for k in range(K):
        wk = aw[:, :, k:k + 1] * feats
        wks = jnp.sum(wk, axis=0)
        acc = acc + jnp.dot(wks, wcat_ref[k], preferred_element_type=jnp.float32)
    outv = acc + bias_ref[...]
    o_ref[...] = jnp.where(outv >= 0.0, outv, 0.1 * outv)


def kernel(q_pts, s_pts, neighb_inds, x, q_lrf, s_lrf, weights, lrf_W, lrf_b, bias):
    table = jnp.concatenate(
        [x, s_lrf.reshape(N, NS * 9), s_pts,
         jnp.zeros((N, TBL_C - IN_CH - NS * 9 - P), jnp.float32)], axis=1)
    flat_idx = jnp.pad(neighb_inds.T.astype(jnp.int32),
                       ((0, 0), (0, N_PAD - N))).reshape(-1)

    rl = q_lrf.reshape(N, NQ * 9)
    rbigs = [jnp.broadcast_to(q_lrf[:, :, None, p, :, None],
                              (N, NQ, NS, P, P)).reshape(N, 36)
             for p in range(P)]
    qd2 = jnp.concatenate([q_pts, rl] + rbigs, axis=1)

    bw = jnp.zeros((2 * NS * 9, IN_CH), jnp.float32)
    bw = bw.at[0:18, 0:64].set(lrf_W).at[18:36, 64:128].set(lrf_W)
    wbot = weights[:, IN_CH:, :]
    wcat = jnp.concatenate(
        [weights[:, :IN_CH, :], jnp.einsum('fe,keo->kfo', bw, wbot)], axis=1)
    b2 = jnp.concatenate([lrf_b, lrf_b])
    bvec = jnp.einsum('e,keo->ko', b2, wbot)

    g = _sc_gather(table, flat_idx)
    gh = g.reshape(H, N_PAD, TBL_C)

    out = pl.pallas_call(
        _tc_body,
        grid=(N // B,),
        in_specs=[
            pl.BlockSpec((H, B, TBL_C), lambda i: (0, i, 0)),
            pl.BlockSpec((B, QD_C), lambda i: (i, 0)),
            pl.BlockSpec((K, 164, 128), lambda i: (0, 0, 0)),
            pl.BlockSpec((K, 128), lambda i: (0, 0)),
            pl.BlockSpec((1, 128), lambda i: (0, 0)),
            pl.BlockSpec((P, 2 * K), lambda i: (0, 0)),
        ],
        out_specs=pl.BlockSpec((B, 128), lambda i: (i, 0)),
        out_shape=jax.ShapeDtypeStruct((N, 128), jnp.float32),
    )(gh, qd2, wcat, bvec, bias[None, :], jnp.asarray(np.stack(_KPR)))
    return out


# speedup vs baseline: 1.5155x; 1.0230x over previous
import functools
import numpy as np
import jax
import jax.numpy as jnp
from jax import lax
from jax.experimental import pallas as pl
from jax.experimental.pallas import tpu as pltpu
from jax.experimental.pallas import tpu_sc as plsc

K = 15
P = 3
IN_CH = 128
NQ = 2
NS = 2
KP_EXTENT = 1.2
RADIUS = 2.5
N = 10000
H = 16
TBL_C = 256
NW = 32
CHUNK = 128
N_PAD = 10240
BPW = 5120
E_PAD = NW * BPW
NCHUNK = BPW // CHUNK
B = 200
QD_C = 21 + 3 * 36


def _kp_np():
    rng = np.random.RandomState(42)
    kp = rng.uniform(-1.0, 1.0, size=(K, P)).astype(np.float32)
    kp = kp / np.maximum(np.linalg.norm(kp, axis=1, keepdims=True), 1e-6)
    kp = kp * rng.uniform(0.3, 1.0, size=(K, 1)).astype(np.float32) * RADIUS
    kp[0, :] = 0.0
    return kp


_KP = _kp_np()
_KPR = [np.concatenate([_KP[:, r], _KP[:, r]]).astype(np.float32) for r in range(P)]


def _sc_gather(table, idx):
    mesh = plsc.VectorSubcoreMesh(core_axis_name="c", subcore_axis_name="s")

    @functools.partial(
        pl.kernel,
        mesh=mesh,
        out_type=jax.ShapeDtypeStruct((E_PAD, TBL_C), jnp.float32),
        scratch_types=[
            pltpu.VMEM((CHUNK,), jnp.int32),
            pltpu.VMEM((CHUNK,), jnp.int32),
            pltpu.VMEM((CHUNK, TBL_C), jnp.float32),
            pltpu.VMEM((CHUNK, TBL_C), jnp.float32),
            pltpu.SemaphoreType.DMA,
            pltpu.SemaphoreType.DMA,
        ],
    )
    def gk(table_hbm, idx_hbm, out_hbm, idx_a, idx_b, rows_a, rows_b, sem_a, sem_b):
        wid = lax.axis_index("s") * 2 + lax.axis_index("c")
        base = wid * BPW

        def fire(idx_v, rows_v, sem, j):
            off = base + j * CHUNK
            pltpu.sync_copy(idx_hbm.at[pl.ds(off, CHUNK)], idx_v)
            pltpu.async_copy(table_hbm.at[idx_v], rows_v, sem)

        def drain(rows_v, sem, j):
            off = base + j * CHUNK
            pltpu.make_async_copy(table_hbm.at[pl.ds(0, CHUNK)], rows_v, sem).wait()
            pltpu.sync_copy(rows_v, out_hbm.at[pl.ds(off, CHUNK)])

        fire(idx_a, rows_a, sem_a, 0)

        def body(jj, carry):
            j0 = jj * 2
            fire(idx_b, rows_b, sem_b, j0 + 1)
            drain(rows_a, sem_a, j0)

            @pl.when(jj < NCHUNK // 2 - 1)
            def _():
                fire(idx_a, rows_a, sem_a, j0 + 2)

            drain(rows_b, sem_b, j0 + 1)
            return carry

        lax.fori_loop(0, NCHUNK // 2, body, 0)

    return gk(table, idx)


def _tc_body(gh_ref, qd_ref, wcat_ref, bvec_ref, bias_ref, kpr_ref, o_ref):
    gh = gh_ref[...]
    qd = qd_ref[...]
    d = gh[:, :, 146:149] - qd[None, :, 0:3]
    als = []
    for q in range(NQ):
        a = d[:, :, 0:1] * qd[None, :, 3 + q * 9 + 0:3 + q * 9 + 3]
        for p in range(1, P):
            a = a + d[:, :, p:p + 1] * qd[None, :, 3 + q * 9 + p * 3:3 + q * 9 + (p + 1) * 3]
        als.append(a)
    al = jnp.concatenate(als, axis=2)
    sq = None
    for r in range(P):
        alx = jnp.concatenate(
            [jnp.broadcast_to(al[:, :, r:r + 1], (H, B, K)),
             jnp.broadcast_to(al[:, :, 3 + r:4 + r], (H, B, K))], axis=2)
        diff = alx - kpr_ref[...][r:r + 1, :][None]
        sq = diff * diff if sq is None else sq + diff * diff
    w = jnp.maximum(1.0 - jnp.sqrt(sq) * (1.0 / KP_EXTENT), 0.0)
    aw = w[:, :, :K] + w[:, :, K:]
    S = gh[:, :, 128:146]
    alrf = None
    for p in range(P):
        pieces = []
        for s in range(NS):
            sl = S[:, :, s * 9 + p * 3:s * 9 + (p + 1) * 3]
            pieces += [sl, sl, sl]
        e18 = jnp.concatenate(pieces, axis=2)
        sbig = jnp.concatenate([e18, e18], axis=2)
        rbig = qd[None, :, 21 + p * 36:21 + (p + 1) * 36]
        alrf = rbig * sbig if alrf is None else alrf + rbig * sbig
    feats = jnp.concatenate([gh[:, :, 0:128], alrf], axis=2)
    acc = jnp.dot(jnp.sum(aw, axis=0), bvec_ref[...],
                  preferred_element_type=jnp.float32)
    for k in range(K):
        wk = aw[:, :, k:k + 1] * feats
        wks = jnp.sum(wk, axis=0)
        acc = acc + jnp.dot(wks, wcat_ref[k], preferred_element_type=jnp.float32)
    outv = acc + bias_ref[...]
    o_ref[...] = jnp.where(outv >= 0.0, outv, 0.1 * outv)


def kernel(q_pts, s_pts, neighb_inds, x, q_lrf, s_lrf, weights, lrf_W, lrf_b, bias):
    table = jnp.concatenate(
        [x, s_lrf.reshape(N, NS * 9), s_pts,
         jnp.zeros((N, TBL_C - IN_CH - NS * 9 - P), jnp.float32)], axis=1)
    flat_idx = jnp.pad(neighb_inds.T.astype(jnp.int32),
                       ((0, 0), (0, N_PAD - N))).reshape(-1)

    rl = q_lrf.reshape(N, NQ * 9)
    rbigs = [jnp.broadcast_to(q_lrf[:, :, None, p, :, None],
                              (N, NQ, NS, P, P)).reshape(N, 36)
             for p in range(P)]
    qd2 = jnp.concatenate([q_pts, rl] + rbigs, axis=1)

    bw = jnp.zeros((2 * NS * 9, IN_CH), jnp.float32)
    bw = bw.at[0:18, 0:64].set(lrf_W).at[18:36, 64:128].set(lrf_W)
    wbot = weights[:, IN_CH:, :]
    wcat = jnp.concatenate(
        [weights[:, :IN_CH, :], jnp.einsum('fe,keo->kfo', bw, wbot)], axis=1)
    b2 = jnp.concatenate([lrf_b, lrf_b])
    bvec = jnp.einsum('e,keo->ko', b2, wbot)

    g = _sc_gather(table, flat_idx)
    gh = g.reshape(H, N_PAD, TBL_C)

    out = pl.pallas_call(
        _tc_body,
        grid=(N // B,),
        in_specs=[
            pl.BlockSpec((H, B, TBL_C), lambda i: (0, i, 0)),
            pl.BlockSpec((B, QD_C), lambda i: (i, 0)),
            pl.BlockSpec((K, 164, 128), lambda i: (0, 0, 0)),
            pl.BlockSpec((K, 128), lambda i: (0, 0)),
            pl.BlockSpec((1, 128), lambda i: (0, 0)),
            pl.BlockSpec((P, 2 * K), lambda i: (0, 0)),
        ],
        out_specs=pl.BlockSpec((B, 128), lambda i: (i, 0)),
        out_shape=jax.ShapeDtypeStruct((N, 128), jnp.float32),
    )(gh, qd2, wcat, bvec, bias[None, :], jnp.asarray(np.stack(_KPR)))
    return out
